# baseline (speedup 1.0000x reference)
import functools

import jax
import jax.numpy as jnp
from jax import lax
from jax.experimental import pallas as pl
from jax.experimental.pallas import tpu as pltpu
from jax.experimental.pallas import tpu_sc as plsc

SEQ_LEN = 200
EMBED = 32
BATCH = 4096
LANES = 16

NC, NS = 2, 16
NW = NC * NS
LT = SEQ_LEN // 8
ROWS_PER_TILE = 8 * 128

_mesh = plsc.VectorSubcoreMesh(core_axis_name="c", subcore_axis_name="s")


@functools.partial(
    pl.kernel,
    out_type=jax.ShapeDtypeStruct((SEQ_LEN, 128, 1024), jnp.float32),
    mesh=_mesh,
    scratch_types=[
        pltpu.VMEM((8, 128), jnp.int32),
        pltpu.VMEM((8, 128), jnp.int32),
        pltpu.VMEM((ROWS_PER_TILE, EMBED), jnp.float32),
        pltpu.VMEM((ROWS_PER_TILE, EMBED), jnp.float32),
        pltpu.VMEM((8, 4096), jnp.float32),
        pltpu.VMEM((SEQ_LEN, EMBED), jnp.float32),
        pltpu.SemaphoreType.DMA,
        pltpu.SemaphoreType.DMA,
        pltpu.SemaphoreType.DMA,
        pltpu.SemaphoreType.DMA,
        pltpu.SemaphoreType.DMA,
    ],
    compiler_params=pltpu.CompilerParams(
        use_tc_tiling_on_sc=False, needs_layout_passes=False),
)
def _emb_kernel(idx_hbm, tok_hbm, pos_hbm, out_hbm,
                idx0, idx1, rows0, rows1, tile_v, pos_v,
                g0, g1, i0, i1, osem):
    wid = lax.axis_index("s") * NC + lax.axis_index("c")
    pltpu.sync_copy(pos_hbm, pos_v)

    idx = (idx0, idx1)
    rows = (rows0, rows1)
    gsem = (g0, g1)
    isem = (i0, i1)
    riota = lax.iota(jnp.int32, LANES)

    def issue(lt, b):
        pltpu.make_async_copy(idx_hbm.at[0, 0], idx[b], isem[b]).wait()

        @pl.loop(0, 8)
        def _(li):
            for v in range(8):
                t = idx[b][li, pl.ds(v * LANES, LANES)]
                r = ((t & -_TBLK)
                     | ((t & (_TSUB - 1)) << 2)
                     | ((t >> _TSH) & 3))
                idx[b][li, pl.ds(v * LANES, LANES)] = r

        for li in range(8):
            pltpu.async_copy(
                tok_hbm.at[idx[b].at[li]],
                rows[b].at[pl.ds(li * 128, 128)],
                gsem[b],
            )

    def complete(lt, b, drain_out):
        pltpu.make_async_copy(
            tok_hbm.at[pl.ds(0, ROWS_PER_TILE)], rows[b], gsem[b]).wait()
        @pl.when(lt + 2 < LT)
        def _():
            pltpu.async_copy(idx_hbm.at[lt + 2, wid], idx[b], isem[b])
        drain_out()

        @pl.loop(0, 8)
        def _(li):
            l = lt * 8 + li
            p0 = pos_v[l, pl.ds(0, LANES)]
            p1 = pos_v[l, pl.ds(LANES, LANES)]
            row_li = jnp.broadcast_to(li, (LANES,))

            @pl.loop(0, LANES, unroll=2)
            def _(k):
                perm = (riota + k) & 15
                for h in range(2):
                    colh = riota + h * LANES
                    base = perm + (riota * 128 + h * 2048)
                    ph = (p0, p1)[h]
                    for v in range(8):
                        ridx = perm + (li * 128 + v * 16)
                        val = plsc.load_gather(rows[b], [ridx, colh])
                        plsc.store_scatter(
                            tile_v, [row_li, base + v * 16], val + ph)

        for dt in range(4):
            pltpu.async_copy(
                tile_v.at[:, pl.ds(dt * 1024, 1024)],
                out_hbm.at[pl.ds(lt * 8, 8), dt * 32 + wid],
                osem,
            )

    def drain_tile():
        for _ in range(4):
            pltpu.make_async_copy(
                tile_v.at[:, pl.ds(0, 1024)],
                out_hbm.at[pl.ds(0, 8), 0], osem).wait()

    pltpu.async_copy(idx_hbm.at[0, wid], idx[0], isem[0])
    pltpu.async_copy(idx_hbm.at[1, wid], idx[1], isem[1])
    issue(0, 0)

    @pl.loop(0, (LT - 1) // 2)
    def _(g):
        issue(2 * g + 1, 1)
        complete(2 * g, 0, lambda: pl.when(g > 0)(drain_tile))
        issue(2 * g + 2, 0)
        complete(2 * g + 1, 1, drain_tile)

    complete(LT - 1, 0, drain_tile)
    drain_tile()


_TBLK = 65536
_TSUB = _TBLK // 4
_TSH = _TSUB.bit_length() - 1
_TGRID = -(-1000000 // _TBLK)


def _transpose_body(x_ref, o_ref):
    eye = jnp.eye(EMBED, dtype=jnp.float32)
    y = lax.dot_general(x_ref[...], eye, (((0,), (0,)), ((), ())),
                        preferred_element_type=jnp.float32)
    for k in range(4):
        o_ref[:, 32 * k:32 * (k + 1)] = y[_TSUB * k:_TSUB * (k + 1), :]


_tok_relayout = pl.pallas_call(
    _transpose_body,
    out_shape=jax.ShapeDtypeStruct((_TGRID * (_TBLK // 4), 128), jnp.float32),
    grid=(_TGRID,),
    in_specs=[pl.BlockSpec((EMBED, _TBLK), lambda i: (0, i))],
    out_specs=pl.BlockSpec((_TBLK // 4, 128), lambda i: (i, 0)),
)


def kernel(inputs, token_table, position_table):
    idx4d = (inputs.astype(jnp.int32).T
             .reshape(LT, 8, 32, 128).transpose(0, 2, 1, 3))
    tok = _tok_relayout(token_table.T).reshape(_TGRID * _TBLK, EMBED)
    out3 = _emb_kernel(idx4d, tok, position_table)
    out5 = out3.reshape(SEQ_LEN, 4, 32, 8, 128)
    return out5.transpose(2, 4, 0, 1, 3).reshape(BATCH, SEQ_LEN, EMBED)


# speedup vs baseline: 1.0123x; 1.0123x over previous
import functools

import jax
import jax.numpy as jnp
from jax import lax
from jax.experimental import pallas as pl
from jax.experimental.pallas import tpu as pltpu
from jax.experimental.pallas import tpu_sc as plsc

SEQ_LEN = 200
EMBED = 32
BATCH = 4096
LANES = 16

NC, NS = 2, 16
NW = NC * NS
LT = SEQ_LEN // 8
ROWS_PER_TILE = 8 * 128

_mesh = plsc.VectorSubcoreMesh(core_axis_name="c", subcore_axis_name="s")


@functools.partial(
    pl.kernel,
    out_type=jax.ShapeDtypeStruct((SEQ_LEN, 128, 1024), jnp.float32),
    mesh=_mesh,
    scratch_types=[
        pltpu.VMEM((8, 128), jnp.int32),
        pltpu.VMEM((8, 128), jnp.int32),
        pltpu.VMEM((ROWS_PER_TILE, EMBED), jnp.float32),
        pltpu.VMEM((ROWS_PER_TILE, EMBED), jnp.float32),
        pltpu.VMEM((8, 4096), jnp.float32),
        pltpu.VMEM((SEQ_LEN, EMBED), jnp.float32),
        pltpu.SemaphoreType.DMA,
        pltpu.SemaphoreType.DMA,
        pltpu.SemaphoreType.DMA,
        pltpu.SemaphoreType.DMA,
        pltpu.SemaphoreType.DMA,
    ],
    compiler_params=pltpu.CompilerParams(
        use_tc_tiling_on_sc=False, needs_layout_passes=False),
)
def _emb_kernel(idx_hbm, tok_hbm, pos_hbm, out_hbm,
                idx0, idx1, rows0, rows1, tile_v, pos_v,
                g0, g1, i0, i1, osem):
    wid = lax.axis_index("s") * NC + lax.axis_index("c")
    pltpu.sync_copy(pos_hbm, pos_v)

    idx = (idx0, idx1)
    rows = (rows0, rows1)
    gsem = (g0, g1)
    isem = (i0, i1)
    riota = lax.iota(jnp.int32, LANES)

    def issue(lt, b):
        pltpu.make_async_copy(idx_hbm.at[0, 0], idx[b], isem[b]).wait()

        @pl.loop(0, 8)
        def _(li):
            for v in range(8):
                t = idx[b][li, pl.ds(v * LANES, LANES)]
                r = ((t & -_TBLK)
                     | ((t & (_TSUB - 1)) << 2)
                     | ((t >> _TSH) & 3))
                idx[b][li, pl.ds(v * LANES, LANES)] = r

        for li in range(8):
            pltpu.async_copy(
                tok_hbm.at[idx[b].at[li]],
                rows[b].at[pl.ds(li * 128, 128)],
                gsem[b],
            )

    def complete(lt, b, drain_out):
        pltpu.make_async_copy(
            tok_hbm.at[pl.ds(0, ROWS_PER_TILE)], rows[b], gsem[b]).wait()
        @pl.when(lt + 2 < LT)
        def _():
            pltpu.async_copy(idx_hbm.at[lt + 2, wid], idx[b], isem[b])
        drain_out()

        @pl.loop(0, 8)
        def _(li):
            l = lt * 8 + li
            p0 = pos_v[l, pl.ds(0, LANES)]
            p1 = pos_v[l, pl.ds(LANES, LANES)]
            row_li = jnp.broadcast_to(li, (LANES,))

            @pl.loop(0, LANES, unroll=2)
            def _(k):
                perm = (riota + k) & 15
                for h in range(2):
                    colh = riota + h * LANES
                    base = perm + (riota * 128 + h * 2048)
                    ph = (p0, p1)[h]
                    for v in range(8):
                        ridx = perm + (li * 128 + v * 16)
                        val = plsc.load_gather(rows[b], [ridx, colh])
                        plsc.store_scatter(
                            tile_v, [row_li, base + v * 16], val + ph)

        for dt in range(4):
            pltpu.async_copy(
                tile_v.at[:, pl.ds(dt * 1024, 1024)],
                out_hbm.at[pl.ds(lt * 8, 8), dt * 32 + wid],
                osem,
            )

    def drain_tile():
        for _ in range(4):
            pltpu.make_async_copy(
                tile_v.at[:, pl.ds(0, 1024)],
                out_hbm.at[pl.ds(0, 8), 0], osem).wait()

    pltpu.async_copy(idx_hbm.at[0, wid], idx[0], isem[0])
    pltpu.async_copy(idx_hbm.at[1, wid], idx[1], isem[1])
    issue(0, 0)

    @pl.loop(0, (LT - 1) // 2)
    def _(g):
        issue(2 * g + 1, 1)
        complete(2 * g, 0, lambda: pl.when(g > 0)(drain_tile))
        issue(2 * g + 2, 0)
        complete(2 * g + 1, 1, drain_tile)

    complete(LT - 1, 0, drain_tile)
    drain_tile()


_TBLK = 16384
_TSUB = _TBLK // 4
_TSH = _TSUB.bit_length() - 1
_TGRID = -(-1000000 // _TBLK)


def _transpose_body(x_ref, o_ref):
    eye = jnp.eye(EMBED, dtype=jnp.float32)
    y = lax.dot_general(x_ref[...], eye, (((0,), (0,)), ((), ())),
                        preferred_element_type=jnp.float32)
    for k in range(4):
        o_ref[:, 32 * k:32 * (k + 1)] = y[_TSUB * k:_TSUB * (k + 1), :]


_tok_relayout = pl.pallas_call(
    _transpose_body,
    out_shape=jax.ShapeDtypeStruct((_TGRID * (_TBLK // 4), 128), jnp.float32),
    grid=(_TGRID,),
    in_specs=[pl.BlockSpec((EMBED, _TBLK), lambda i: (0, i))],
    out_specs=pl.BlockSpec((_TBLK // 4, 128), lambda i: (i, 0)),
)


def kernel(inputs, token_table, position_table):
    idx4d = (inputs.astype(jnp.int32).T
             .reshape(LT, 8, 32, 128).transpose(0, 2, 1, 3))
    tok = _tok_relayout(token_table.T).reshape(_TGRID * _TBLK, EMBED)
    out3 = _emb_kernel(idx4d, tok, position_table)
    out5 = out3.reshape(SEQ_LEN, 4, 32, 8, 128)
    return out5.transpose(2, 4, 0, 1, 3).reshape(BATCH, SEQ_LEN, EMBED)


# speedup vs baseline: 1.5732x; 1.5541x over previous
import functools

import jax
import jax.numpy as jnp
from jax import lax
from jax.experimental import pallas as pl
from jax.experimental.pallas import tpu as pltpu
from jax.experimental.pallas import tpu_sc as plsc

SEQ_LEN = 200
EMBED = 32
BATCH = 4096
LANES = 16

NC, NS = 2, 16
NW = NC * NS
LT = SEQ_LEN // 8
ROWS_PER_TILE = 8 * 128

_mesh = plsc.VectorSubcoreMesh(core_axis_name="c", subcore_axis_name="s")


@functools.partial(
    pl.kernel,
    out_type=jax.ShapeDtypeStruct((SEQ_LEN, 128, 1024), jnp.float32),
    mesh=_mesh,
    scratch_types=[
        pltpu.VMEM((8, 128), jnp.int32),
        pltpu.VMEM((8, 128), jnp.int32),
        pltpu.VMEM((ROWS_PER_TILE, EMBED), jnp.float32),
        pltpu.VMEM((ROWS_PER_TILE, EMBED), jnp.float32),
        pltpu.VMEM((8, 4096), jnp.float32),
        pltpu.VMEM((SEQ_LEN, EMBED), jnp.float32),
        pltpu.SemaphoreType.DMA,
        pltpu.SemaphoreType.DMA,
        pltpu.SemaphoreType.DMA,
        pltpu.SemaphoreType.DMA,
        pltpu.SemaphoreType.DMA,
    ],
    compiler_params=pltpu.CompilerParams(
        use_tc_tiling_on_sc=False, needs_layout_passes=False),
)
def _emb_kernel(idx_hbm, tok_hbm, pos_hbm, out_hbm,
                idx0, idx1, rows0, rows1, tile_v, pos_v,
                g0, g1, i0, i1, osem):
    wid = lax.axis_index("s") * NC + lax.axis_index("c")
    pltpu.sync_copy(pos_hbm, pos_v)

    idx = (idx0, idx1)
    rows = (rows0, rows1)
    gsem = (g0, g1)
    isem = (i0, i1)
    riota = lax.iota(jnp.int32, LANES)

    def issue(lt, b):
        pltpu.make_async_copy(idx_hbm.at[0, 0], idx[b], isem[b]).wait()

        @pl.loop(0, 8)
        def _(li):
            for v in range(8):
                t = idx[b][li, pl.ds(v * LANES, LANES)]
                r = ((t & -_TBLK)
                     | ((t & (_TSUB - 1)) << 2)
                     | ((t >> _TSH) & 3))
                idx[b][li, pl.ds(v * LANES, LANES)] = r

        for li in range(8):
            pltpu.async_copy(
                tok_hbm.at[idx[b].at[li]],
                rows[b].at[pl.ds(li * 128, 128)],
                gsem[b],
            )

    def complete(lt, b, drain_out):
        pltpu.make_async_copy(
            tok_hbm.at[pl.ds(0, ROWS_PER_TILE)], rows[b], gsem[b]).wait()
        @pl.when(lt + 2 < LT)
        def _():
            pltpu.async_copy(idx_hbm.at[lt + 2, wid], idx[b], isem[b])
        drain_out()

        zv = jnp.zeros((LANES,), jnp.int32)

        @pl.loop(0, 8)
        def _(li):
            l = lt * 8 + li
            p0 = pos_v[l, pl.ds(0, LANES)]
            p1 = pos_v[l, pl.ds(LANES, LANES)]
            lrow = li * 4096

            @plsc.parallel_loop(0, LANES, unroll=2)
            def _(k):
                perm = (riota + k) & 15
                p32 = perm << 5
                for h in range(2):
                    gb = p32 + (riota + h * 16)
                    sb = perm + (riota * 128 + h * 2048)
                    ph = (p0, p1)[h]
                    for v in range(8):
                        val = plsc.load_gather(
                            rows[b], [zv, gb + (lrow + v * 512)])
                        plsc.store_scatter(
                            tile_v, [zv, sb + (lrow + v * 16)], val + ph)

        for dt in range(4):
            pltpu.async_copy(
                tile_v.at[:, pl.ds(dt * 1024, 1024)],
                out_hbm.at[pl.ds(lt * 8, 8), dt * 32 + wid],
                osem,
            )

    def drain_tile():
        for _ in range(4):
            pltpu.make_async_copy(
                tile_v.at[:, pl.ds(0, 1024)],
                out_hbm.at[pl.ds(0, 8), 0], osem).wait()

    pltpu.async_copy(idx_hbm.at[0, wid], idx[0], isem[0])
    pltpu.async_copy(idx_hbm.at[1, wid], idx[1], isem[1])
    issue(0, 0)

    @pl.loop(0, (LT - 1) // 2)
    def _(g):
        issue(2 * g + 1, 1)
        complete(2 * g, 0, lambda: pl.when(g > 0)(drain_tile))
        issue(2 * g + 2, 0)
        complete(2 * g + 1, 1, drain_tile)

    complete(LT - 1, 0, drain_tile)
    drain_tile()


_TBLK = 16384
_TSUB = _TBLK // 4
_TSH = _TSUB.bit_length() - 1
_TGRID = -(-1000000 // _TBLK)


def _transpose_body(x_ref, o_ref):
    eye = jnp.eye(EMBED, dtype=jnp.float32)
    y = lax.dot_general(x_ref[...], eye, (((0,), (0,)), ((), ())),
                        preferred_element_type=jnp.float32)
    for k in range(4):
        o_ref[:, 32 * k:32 * (k + 1)] = y[_TSUB * k:_TSUB * (k + 1), :]


_tok_relayout = pl.pallas_call(
    _transpose_body,
    out_shape=jax.ShapeDtypeStruct((_TGRID * (_TBLK // 4), 128), jnp.float32),
    grid=(_TGRID,),
    in_specs=[pl.BlockSpec((EMBED, _TBLK), lambda i: (0, i))],
    out_specs=pl.BlockSpec((_TBLK // 4, 128), lambda i: (i, 0)),
)


def kernel(inputs, token_table, position_table):
    idx4d = (inputs.astype(jnp.int32).T
             .reshape(LT, 8, 32, 128).transpose(0, 2, 1, 3))
    tok = _tok_relayout(token_table.T).reshape(_TGRID * _TBLK, EMBED)
    out3 = _emb_kernel(idx4d, tok, position_table)
    out5 = out3.reshape(SEQ_LEN, 4, 32, 8, 128)
    return out5.transpose(2, 4, 0, 1, 3).reshape(BATCH, SEQ_LEN, EMBED)


# speedup vs baseline: 2.6503x; 1.6846x over previous
import functools

import jax
import jax.numpy as jnp
from jax import lax
from jax.experimental import pallas as pl
from jax.experimental.pallas import tpu as pltpu
from jax.experimental.pallas import tpu_sc as plsc

SEQ_LEN = 200
EMBED = 32
BATCH = 4096
LANES = 16

NC, NS = 2, 16
NW = NC * NS
LT = SEQ_LEN // 8
ROWS_PER_TILE = 8 * 128

_mesh = plsc.VectorSubcoreMesh(core_axis_name="c", subcore_axis_name="s")


@functools.partial(
    pl.kernel,
    out_type=jax.ShapeDtypeStruct((SEQ_LEN, 128, 1024), jnp.float32),
    mesh=_mesh,
    scratch_types=[
        pltpu.VMEM((8, 128), jnp.int32),
        pltpu.VMEM((8, 128), jnp.int32),
        pltpu.VMEM((ROWS_PER_TILE, EMBED), jnp.float32),
        pltpu.VMEM((ROWS_PER_TILE, EMBED), jnp.float32),
        pltpu.VMEM((8, 4096), jnp.float32),
        pltpu.VMEM((SEQ_LEN, EMBED), jnp.float32),
        pltpu.SemaphoreType.DMA,
        pltpu.SemaphoreType.DMA,
        pltpu.SemaphoreType.DMA,
        pltpu.SemaphoreType.DMA,
        pltpu.SemaphoreType.DMA,
    ],
    compiler_params=pltpu.CompilerParams(
        use_tc_tiling_on_sc=False, needs_layout_passes=False),
)
def _emb_kernel(idx_hbm, tok_hbm, pos_hbm, out_hbm,
                idx0, idx1, rows0, rows1, tile_v, pos_v,
                g0, g1, i0, i1, osem):
    wid = lax.axis_index("s") * NC + lax.axis_index("c")
    pltpu.sync_copy(pos_hbm, pos_v)

    idx = (idx0, idx1)
    rows = (rows0, rows1)
    gsem = (g0, g1)
    isem = (i0, i1)
    riota = lax.iota(jnp.int32, LANES)

    def issue(lt, b):
        pltpu.make_async_copy(idx_hbm.at[0, 0], idx[b], isem[b]).wait()

        @pl.loop(0, 8)
        def _(li):
            for v in range(8):
                t = idx[b][li, pl.ds(v * LANES, LANES)]
                r = ((t & -_PBLK)
                     | ((t & (_PSUB - 1)) << 2)
                     | ((t >> _PSH) & 3))
                idx[b][li, pl.ds(v * LANES, LANES)] = r

        for li in range(8):
            pltpu.async_copy(
                tok_hbm.at[idx[b].at[li]],
                rows[b].at[pl.ds(li * 128, 128)],
                gsem[b],
            )

    def complete(lt, b, drain_out):
        pltpu.make_async_copy(
            tok_hbm.at[pl.ds(0, ROWS_PER_TILE)], rows[b], gsem[b]).wait()
        @pl.when(lt + 2 < LT)
        def _():
            pltpu.async_copy(idx_hbm.at[lt + 2, wid], idx[b], isem[b])
        drain_out()

        zv = jnp.zeros((LANES,), jnp.int32)

        @pl.loop(0, 8)
        def _(li):
            l = lt * 8 + li
            p0 = pos_v[l, pl.ds(0, LANES)]
            p1 = pos_v[l, pl.ds(LANES, LANES)]
            lrow = li * 4096

            @plsc.parallel_loop(0, LANES, unroll=2)
            def _(k):
                perm = (riota + k) & 15
                p32 = perm << 5
                for h in range(2):
                    gb = p32 + (riota + h * 16)
                    sb = perm + (riota * 128 + h * 2048)
                    ph = (p0, p1)[h]
                    for v in range(8):
                        val = plsc.load_gather(
                            rows[b], [zv, gb + (lrow + v * 512)])
                        plsc.store_scatter(
                            tile_v, [zv, sb + (lrow + v * 16)], val + ph)

        for dt in range(4):
            pltpu.async_copy(
                tile_v.at[:, pl.ds(dt * 1024, 1024)],
                out_hbm.at[pl.ds(lt * 8, 8), dt * 32 + wid],
                osem,
            )

    def drain_tile():
        for _ in range(4):
            pltpu.make_async_copy(
                tile_v.at[:, pl.ds(0, 1024)],
                out_hbm.at[pl.ds(0, 8), 0], osem).wait()

    pltpu.async_copy(idx_hbm.at[0, wid], idx[0], isem[0])
    pltpu.async_copy(idx_hbm.at[1, wid], idx[1], isem[1])
    issue(0, 0)

    @pl.loop(0, (LT - 1) // 2)
    def _(g):
        issue(2 * g + 1, 1)
        complete(2 * g, 0, lambda: pl.when(g > 0)(drain_tile))
        issue(2 * g + 2, 0)
        complete(2 * g + 1, 1, drain_tile)

    complete(LT - 1, 0, drain_tile)
    drain_tile()


_TBLK = 16384
_TGRID = -(-1000000 // _TBLK)
_PBLK = 8192
_PSUB = 2048
_PSH = 11


def _transpose_body(x_ref, o_ref):
    x = x_ref[...]
    x256 = jnp.concatenate(
        [x[:, 2048 * c:2048 * (c + 1)] for c in range(8)], axis=0)
    eye = jnp.eye(256, dtype=jnp.float32)
    y = lax.dot_general(x256, eye, (((0,), (0,)), ((), ())),
                        preferred_element_type=jnp.float32)
    for g in range(2):
        o_ref[2048 * g:2048 * (g + 1), :] = y[:, 128 * g:128 * (g + 1)]


_tok_relayout = pl.pallas_call(
    _transpose_body,
    out_shape=jax.ShapeDtypeStruct((_TGRID * (_TBLK // 4), 128), jnp.float32),
    grid=(_TGRID,),
    in_specs=[pl.BlockSpec((EMBED, _TBLK), lambda i: (0, i))],
    out_specs=pl.BlockSpec((_TBLK // 4, 128), lambda i: (i, 0)),
)


def kernel(inputs, token_table, position_table):
    idx4d = (inputs.astype(jnp.int32).T
             .reshape(LT, 8, 32, 128).transpose(0, 2, 1, 3))
    tok = _tok_relayout(token_table.T).reshape(_TGRID * _TBLK, EMBED)
    out3 = _emb_kernel(idx4d, tok, position_table)
    out5 = out3.reshape(SEQ_LEN, 4, 32, 8, 128)
    return out5.transpose(2, 4, 0, 1, 3).reshape(BATCH, SEQ_LEN, EMBED)
